# baseline (speedup 1.0000x reference)
import functools

import jax
import jax.numpy as jnp
from jax import lax
from jax.experimental import pallas as pl
from jax.experimental.pallas import tpu as pltpu
from jax.experimental.pallas import tpu_sc as plsc

B = 16384
D = 64
N = 100000
NP = N // 2
NC = 2
NS = 16
NW = NC * NS
L = 16
NBLK = N // 128
NMAIN = NBLK * 128
NTAILP = (N - NMAIN) // 2
CHUNK = 128
BPW = B // NW
NCH = BPW // CHUNK

_mesh = plsc.VectorSubcoreMesh(core_axis_name="c", subcore_axis_name="s")


def _wid():
    return lax.axis_index("s") * NC + lax.axis_index("c")


@functools.partial(
    pl.kernel,
    mesh=_mesh,
    out_type=jax.ShapeDtypeStruct((NP, 128), jnp.float32),
    compiler_params=pltpu.CompilerParams(needs_layout_passes=False),
    scratch_types=[
        pltpu.VMEM((2, 64, 128), jnp.float32),
        pltpu.VMEM((2, 64, 128), jnp.float32),
        pltpu.VMEM((NTAILP, 128), jnp.float32),
        pltpu.SemaphoreType.DMA,
        pltpu.SemaphoreType.DMA,
        pltpu.SemaphoreType.DMA,
        pltpu.SemaphoreType.DMA,
    ],
)
def _transpose_pairs(tableT_hbm, aux_hbm, s_hbm, blk_v, stage_v, tail_v,
                     isem0, isem1, osem0, osem1):
    wid = _wid()
    lo = wid * NBLK // NW
    hi = (wid + 1) * NBLK // NW
    isems = [isem0, isem1]
    osems = [osem0, osem1]
    iq = [lax.iota(jnp.int32, L) + k * L for k in range(4)]
    zero = iq[0] * 0

    def fetch(beta, buf):
        return pltpu.async_copy(
            tableT_hbm.at[:, pl.ds(beta * 128, 128)], blk_v.at[buf],
            isems[buf])

    def flush(beta, buf):
        return pltpu.async_copy(
            stage_v.at[buf], s_hbm.at[pl.ds(beta * 64, 64), :], osems[buf])

    def transpose_block(buf):
        @plsc.parallel_loop(0, 64, step=1, unroll=16)
        def pair_body(p):
            for h in range(2):
                il = zero + (2 * p + h)
                for k in range(4):
                    vals = plsc.load_gather(blk_v.at[buf], [iq[k], il])
                    stage_v[buf, p, pl.ds(h * 64 + k * L, L)] = vals

    nblocks = hi - lo

    fetch(lo, 0)

    def outer(i2, carry):
        beta0 = lo + 2 * i2
        @pl.when(beta0 + 1 < hi)
        def _():
            fetch(beta0 + 1, 1)

        pltpu.make_async_copy(
            tableT_hbm.at[:, pl.ds(beta0 * 128, 128)], blk_v.at[0],
            isems[0]).wait()
        @pl.when(i2 > 0)
        def _():
            pltpu.make_async_copy(
                stage_v.at[0], s_hbm.at[pl.ds(beta0 * 64, 64), :],
                osems[0]).wait()
        transpose_block(0)
        flush(beta0, 0)

        @pl.when(beta0 + 2 < hi)
        def _():
            fetch(beta0 + 2, 0)

        @pl.when(beta0 + 1 < hi)
        def _():
            pltpu.make_async_copy(
                tableT_hbm.at[:, pl.ds((beta0 + 1) * 128, 128)], blk_v.at[1],
                isems[1]).wait()
            @pl.when(i2 > 0)
            def _():
                pltpu.make_async_copy(
                    stage_v.at[1], s_hbm.at[pl.ds((beta0 + 1) * 64, 64), :],
                    osems[1]).wait()
            transpose_block(1)
            flush(beta0 + 1, 1)

        return carry

    lax.fori_loop(0, (nblocks + 1) // 2, outer, 0)

    pltpu.make_async_copy(
        stage_v.at[0], s_hbm.at[pl.ds(lo * 64, 64), :], osems[0]).wait()
    @pl.when(nblocks > 1)
    def _():
        pltpu.make_async_copy(
            stage_v.at[1], s_hbm.at[pl.ds(lo * 64, 64), :], osems[1]).wait()

    @pl.when(wid == NW - 1)
    def _tail():
        pltpu.sync_copy(aux_hbm, tail_v)
        pltpu.sync_copy(tail_v, s_hbm.at[pl.ds(NMAIN // 2, NTAILP), :])


@functools.partial(
    pl.kernel,
    mesh=_mesh,
    out_type=jax.ShapeDtypeStruct((D, B), jnp.float32),
    compiler_params=pltpu.CompilerParams(needs_layout_passes=False),
    scratch_types=[
        pltpu.VMEM((NCH, CHUNK), jnp.int32),
        pltpu.VMEM((NCH, CHUNK), jnp.int32),
        pltpu.VMEM((NCH, CHUNK), jnp.int32),
        pltpu.VMEM((NCH, CHUNK, 128), jnp.float32),
        pltpu.VMEM((NCH, D, CHUNK), jnp.float32),
        pltpu.SemaphoreType.DMA,
        pltpu.SemaphoreType.DMA,
        pltpu.SemaphoreType.DMA,
        pltpu.SemaphoreType.DMA,
        pltpu.SemaphoreType.DMA,
        pltpu.SemaphoreType.DMA,
    ],
)
def _gather_add(s_hbm, headT_hbm, idx_hbm, outT_hbm,
                idx_v, pidx_v, cb_v, g_v, h_v,
                gs0, gs1, gs2, gs3, hsem, osem):
    wid = _wid()
    base = wid * BPW
    gsems = [gs0, gs1, gs2, gs3]
    iota = lax.iota(jnp.int32, L)
    zero = iota * 0
    rows = [iota + g * L for g in range(CHUNK // L)]

    for k in range(NCH):
        pltpu.sync_copy(idx_hbm.at[pl.ds(base + k * CHUNK, CHUNK)],
                        idx_v.at[k])

    def prep(i, carry):
        for k in range(NCH):
            v = idx_v[k, pl.ds(i * L, L)]
            pidx_v[k, pl.ds(i * L, L)] = lax.shift_right_logical(v, 1)
            cb_v[k, pl.ds(i * L, L)] = (v & 1) * 64
        return carry

    lax.fori_loop(0, CHUNK // L, prep, 0)

    gcp = [pltpu.async_copy(s_hbm.at[pidx_v.at[k]], g_v.at[k], gsems[k])
           for k in range(NCH)]
    hcp = [pltpu.async_copy(headT_hbm.at[:, pl.ds(base + k * CHUNK, CHUNK)],
                            h_v.at[k], hsem)
           for k in range(NCH)]
    for cp in hcp:
        cp.wait()

    ocp = []
    for k in range(NCH):
        gcp[k].wait()
        cbs = [cb_v[k, pl.ds(g * L, L)] for g in range(CHUNK // L)]

        @plsc.parallel_loop(0, D, step=1, unroll=8)
        def cbody(c, k=k, cbs=cbs):
            for g in range(CHUNK // L):
                vals = plsc.load_gather(g_v.at[k], [rows[g], cbs[g] + c])
                h_v[k, c, pl.ds(g * L, L)] = (
                    h_v[k, c, pl.ds(g * L, L)] + vals)
        ocp.append(pltpu.async_copy(
            h_v.at[k], outT_hbm.at[:, pl.ds(base + k * CHUNK, CHUNK)], osem))

    for cp in ocp:
        cp.wait()


def kernel(head_embed, rel_ids, embed_table):
    aux = embed_table[NMAIN:].reshape(NTAILP, 128)
    s = _transpose_pairs(embed_table.T, aux)
    out_t = _gather_add(s, head_embed.T, rel_ids)
    return out_t.T


# speedup vs baseline: 1.0231x; 1.0231x over previous
import functools

import jax
import jax.numpy as jnp
from jax import lax
from jax.experimental import pallas as pl
from jax.experimental.pallas import tpu as pltpu
from jax.experimental.pallas import tpu_sc as plsc

B = 16384
D = 64
N = 100000
NP = N // 2
NC = 2
NS = 16
NW = NC * NS
L = 16
NBLK = N // 128
NMAIN = NBLK * 128
NTAILP = (N - NMAIN) // 2
CHUNK = 128
BPW = B // NW
NCH = BPW // CHUNK

_mesh = plsc.VectorSubcoreMesh(core_axis_name="c", subcore_axis_name="s")


def _wid():
    return lax.axis_index("s") * NC + lax.axis_index("c")


@functools.partial(
    pl.kernel,
    mesh=_mesh,
    out_type=jax.ShapeDtypeStruct((NP, 128), jnp.float32),
    compiler_params=pltpu.CompilerParams(needs_layout_passes=False),
    scratch_types=[
        pltpu.VMEM((2, 64, 128), jnp.float32),
        pltpu.VMEM((2, 64, 128), jnp.float32),
        pltpu.VMEM((NTAILP, 128), jnp.float32),
        pltpu.SemaphoreType.DMA,
        pltpu.SemaphoreType.DMA,
        pltpu.SemaphoreType.DMA,
        pltpu.SemaphoreType.DMA,
    ],
)
def _transpose_pairs(tableT_hbm, aux_hbm, s_hbm, blk_v, stage_v, tail_v,
                     isem0, isem1, osem0, osem1):
    wid = _wid()
    lo = wid * NBLK // NW
    hi = (wid + 1) * NBLK // NW
    isems = [isem0, isem1]
    osems = [osem0, osem1]
    iq = [lax.iota(jnp.int32, L) + k * L for k in range(4)]
    zero = iq[0] * 0

    def fetch(beta, buf):
        return pltpu.async_copy(
            tableT_hbm.at[:, pl.ds(beta * 128, 128)], blk_v.at[buf],
            isems[buf])

    def flush(beta, buf):
        return pltpu.async_copy(
            stage_v.at[buf], s_hbm.at[pl.ds(beta * 64, 64), :], osems[buf])

    def transpose_block(buf):
        @plsc.parallel_loop(0, 64, step=1, unroll=4)
        def pair_body(p):
            for h in range(2):
                il = zero + (2 * p + h)
                for k in range(4):
                    vals = plsc.load_gather(blk_v.at[buf], [iq[k], il])
                    stage_v[buf, p, pl.ds(h * 64 + k * L, L)] = vals

    nblocks = hi - lo

    fetch(lo, 0)

    def outer(i2, carry):
        beta0 = lo + 2 * i2
        @pl.when(beta0 + 1 < hi)
        def _():
            fetch(beta0 + 1, 1)

        pltpu.make_async_copy(
            tableT_hbm.at[:, pl.ds(beta0 * 128, 128)], blk_v.at[0],
            isems[0]).wait()
        @pl.when(i2 > 0)
        def _():
            pltpu.make_async_copy(
                stage_v.at[0], s_hbm.at[pl.ds(beta0 * 64, 64), :],
                osems[0]).wait()
        transpose_block(0)
        flush(beta0, 0)

        @pl.when(beta0 + 2 < hi)
        def _():
            fetch(beta0 + 2, 0)

        @pl.when(beta0 + 1 < hi)
        def _():
            pltpu.make_async_copy(
                tableT_hbm.at[:, pl.ds((beta0 + 1) * 128, 128)], blk_v.at[1],
                isems[1]).wait()
            @pl.when(i2 > 0)
            def _():
                pltpu.make_async_copy(
                    stage_v.at[1], s_hbm.at[pl.ds((beta0 + 1) * 64, 64), :],
                    osems[1]).wait()
            transpose_block(1)
            flush(beta0 + 1, 1)

        return carry

    lax.fori_loop(0, (nblocks + 1) // 2, outer, 0)

    pltpu.make_async_copy(
        stage_v.at[0], s_hbm.at[pl.ds(lo * 64, 64), :], osems[0]).wait()
    @pl.when(nblocks > 1)
    def _():
        pltpu.make_async_copy(
            stage_v.at[1], s_hbm.at[pl.ds(lo * 64, 64), :], osems[1]).wait()

    @pl.when(wid == NW - 1)
    def _tail():
        pltpu.sync_copy(aux_hbm, tail_v)
        pltpu.sync_copy(tail_v, s_hbm.at[pl.ds(NMAIN // 2, NTAILP), :])


@functools.partial(
    pl.kernel,
    mesh=_mesh,
    out_type=jax.ShapeDtypeStruct((D, B), jnp.float32),
    compiler_params=pltpu.CompilerParams(needs_layout_passes=False),
    scratch_types=[
        pltpu.VMEM((NCH, CHUNK), jnp.int32),
        pltpu.VMEM((NCH, CHUNK), jnp.int32),
        pltpu.VMEM((NCH, CHUNK), jnp.int32),
        pltpu.VMEM((NCH, CHUNK, 128), jnp.float32),
        pltpu.VMEM((NCH, D, CHUNK), jnp.float32),
        pltpu.SemaphoreType.DMA,
        pltpu.SemaphoreType.DMA,
        pltpu.SemaphoreType.DMA,
        pltpu.SemaphoreType.DMA,
        pltpu.SemaphoreType.DMA,
        pltpu.SemaphoreType.DMA,
    ],
)
def _gather_add(s_hbm, headT_hbm, idx_hbm, outT_hbm,
                idx_v, pidx_v, cb_v, g_v, h_v,
                gs0, gs1, gs2, gs3, hsem, osem):
    wid = _wid()
    base = wid * BPW
    gsems = [gs0, gs1, gs2, gs3]
    iota = lax.iota(jnp.int32, L)
    zero = iota * 0
    rows = [iota + g * L for g in range(CHUNK // L)]

    for k in range(NCH):
        pltpu.sync_copy(idx_hbm.at[pl.ds(base + k * CHUNK, CHUNK)],
                        idx_v.at[k])

    def prep(i, carry):
        for k in range(NCH):
            v = idx_v[k, pl.ds(i * L, L)]
            pidx_v[k, pl.ds(i * L, L)] = lax.shift_right_logical(v, 1)
            cb_v[k, pl.ds(i * L, L)] = (v & 1) * 64
        return carry

    lax.fori_loop(0, CHUNK // L, prep, 0)

    gcp = [pltpu.async_copy(s_hbm.at[pidx_v.at[k]], g_v.at[k], gsems[k])
           for k in range(NCH)]
    hcp = [pltpu.async_copy(headT_hbm.at[:, pl.ds(base + k * CHUNK, CHUNK)],
                            h_v.at[k], hsem)
           for k in range(NCH)]
    for cp in hcp:
        cp.wait()

    ocp = []
    for k in range(NCH):
        gcp[k].wait()
        cbs = [cb_v[k, pl.ds(g * L, L)] for g in range(CHUNK // L)]

        @plsc.parallel_loop(0, D, step=1, unroll=8)
        def cbody(c, k=k, cbs=cbs):
            for g in range(CHUNK // L):
                vals = plsc.load_gather(g_v.at[k], [rows[g], cbs[g] + c])
                h_v[k, c, pl.ds(g * L, L)] = (
                    h_v[k, c, pl.ds(g * L, L)] + vals)
        ocp.append(pltpu.async_copy(
            h_v.at[k], outT_hbm.at[:, pl.ds(base + k * CHUNK, CHUNK)], osem))

    for cp in ocp:
        cp.wait()


def kernel(head_embed, rel_ids, embed_table):
    aux = embed_table[NMAIN:].reshape(NTAILP, 128)
    s = _transpose_pairs(embed_table.T, aux)
    out_t = _gather_add(s, head_embed.T, rel_ids)
    return out_t.T


# speedup vs baseline: 1.0242x; 1.0011x over previous
import functools

import jax
import jax.numpy as jnp
from jax import lax
from jax.experimental import pallas as pl
from jax.experimental.pallas import tpu as pltpu
from jax.experimental.pallas import tpu_sc as plsc

B = 16384
D = 64
N = 100000
NP = N // 2
NC = 2
NS = 16
NW = NC * NS
L = 16
NBLK = N // 128
NMAIN = NBLK * 128
NTAILP = (N - NMAIN) // 2
CHUNK = 128
BPW = B // NW
NCH = BPW // CHUNK

_mesh = plsc.VectorSubcoreMesh(core_axis_name="c", subcore_axis_name="s")


def _wid():
    return lax.axis_index("s") * NC + lax.axis_index("c")


@functools.partial(
    pl.kernel,
    mesh=_mesh,
    out_type=jax.ShapeDtypeStruct((NP, 128), jnp.float32),
    compiler_params=pltpu.CompilerParams(needs_layout_passes=False),
    scratch_types=[
        pltpu.VMEM((2, 64, 128), jnp.float32),
        pltpu.VMEM((2, 64, 128), jnp.float32),
        pltpu.VMEM((NTAILP, 128), jnp.float32),
        pltpu.SemaphoreType.DMA,
        pltpu.SemaphoreType.DMA,
        pltpu.SemaphoreType.DMA,
        pltpu.SemaphoreType.DMA,
    ],
)
def _transpose_pairs(tableT_hbm, aux_hbm, s_hbm, blk_v, stage_v, tail_v,
                     isem0, isem1, osem0, osem1):
    wid = _wid()
    lo = wid * NBLK // NW
    hi = (wid + 1) * NBLK // NW
    isems = [isem0, isem1]
    osems = [osem0, osem1]
    iq = [lax.iota(jnp.int32, L) + k * L for k in range(4)]
    zero = iq[0] * 0

    def fetch(beta, buf):
        return pltpu.async_copy(
            tableT_hbm.at[:, pl.ds(beta * 128, 128)], blk_v.at[buf],
            isems[buf])

    def flush(beta, buf):
        return pltpu.async_copy(
            stage_v.at[buf], s_hbm.at[pl.ds(beta * 64, 64), :], osems[buf])

    def transpose_block(buf):
        @plsc.parallel_loop(0, 64, step=1, unroll=8)
        def pair_body(p):
            for h in range(2):
                il = zero + (2 * p + h)
                for k in range(4):
                    vals = plsc.load_gather(blk_v.at[buf], [iq[k], il])
                    stage_v[buf, p, pl.ds(h * 64 + k * L, L)] = vals

    nblocks = hi - lo

    fetch(lo, 0)

    def outer(i2, carry):
        beta0 = lo + 2 * i2
        @pl.when(beta0 + 1 < hi)
        def _():
            fetch(beta0 + 1, 1)

        pltpu.make_async_copy(
            tableT_hbm.at[:, pl.ds(beta0 * 128, 128)], blk_v.at[0],
            isems[0]).wait()
        @pl.when(i2 > 0)
        def _():
            pltpu.make_async_copy(
                stage_v.at[0], s_hbm.at[pl.ds(beta0 * 64, 64), :],
                osems[0]).wait()
        transpose_block(0)
        flush(beta0, 0)

        @pl.when(beta0 + 2 < hi)
        def _():
            fetch(beta0 + 2, 0)

        @pl.when(beta0 + 1 < hi)
        def _():
            pltpu.make_async_copy(
                tableT_hbm.at[:, pl.ds((beta0 + 1) * 128, 128)], blk_v.at[1],
                isems[1]).wait()
            @pl.when(i2 > 0)
            def _():
                pltpu.make_async_copy(
                    stage_v.at[1], s_hbm.at[pl.ds((beta0 + 1) * 64, 64), :],
                    osems[1]).wait()
            transpose_block(1)
            flush(beta0 + 1, 1)

        return carry

    lax.fori_loop(0, (nblocks + 1) // 2, outer, 0)

    pltpu.make_async_copy(
        stage_v.at[0], s_hbm.at[pl.ds(lo * 64, 64), :], osems[0]).wait()
    @pl.when(nblocks > 1)
    def _():
        pltpu.make_async_copy(
            stage_v.at[1], s_hbm.at[pl.ds(lo * 64, 64), :], osems[1]).wait()

    @pl.when(wid == NW - 1)
    def _tail():
        pltpu.sync_copy(aux_hbm, tail_v)
        pltpu.sync_copy(tail_v, s_hbm.at[pl.ds(NMAIN // 2, NTAILP), :])


@functools.partial(
    pl.kernel,
    mesh=_mesh,
    out_type=jax.ShapeDtypeStruct((D, B), jnp.float32),
    compiler_params=pltpu.CompilerParams(needs_layout_passes=False),
    scratch_types=[
        pltpu.VMEM((NCH, CHUNK), jnp.int32),
        pltpu.VMEM((NCH, CHUNK), jnp.int32),
        pltpu.VMEM((NCH, CHUNK), jnp.int32),
        pltpu.VMEM((NCH, CHUNK, 128), jnp.float32),
        pltpu.VMEM((NCH, D, CHUNK), jnp.float32),
        pltpu.SemaphoreType.DMA,
        pltpu.SemaphoreType.DMA,
        pltpu.SemaphoreType.DMA,
        pltpu.SemaphoreType.DMA,
        pltpu.SemaphoreType.DMA,
        pltpu.SemaphoreType.DMA,
    ],
)
def _gather_add(s_hbm, headT_hbm, idx_hbm, outT_hbm,
                idx_v, pidx_v, cb_v, g_v, h_v,
                gs0, gs1, gs2, gs3, hsem, osem):
    wid = _wid()
    base = wid * BPW
    gsems = [gs0, gs1, gs2, gs3]
    iota = lax.iota(jnp.int32, L)
    zero = iota * 0
    rows = [iota + g * L for g in range(CHUNK // L)]

    for k in range(NCH):
        pltpu.sync_copy(idx_hbm.at[pl.ds(base + k * CHUNK, CHUNK)],
                        idx_v.at[k])

    def prep(i, carry):
        for k in range(NCH):
            v = idx_v[k, pl.ds(i * L, L)]
            pidx_v[k, pl.ds(i * L, L)] = lax.shift_right_logical(v, 1)
            cb_v[k, pl.ds(i * L, L)] = (v & 1) * 64
        return carry

    lax.fori_loop(0, CHUNK // L, prep, 0)

    gcp = [pltpu.async_copy(s_hbm.at[pidx_v.at[k]], g_v.at[k], gsems[k])
           for k in range(NCH)]
    hcp = [pltpu.async_copy(headT_hbm.at[:, pl.ds(base + k * CHUNK, CHUNK)],
                            h_v.at[k], hsem)
           for k in range(NCH)]
    for cp in hcp:
        cp.wait()

    ocp = []
    for k in range(NCH):
        gcp[k].wait()
        cbs = [cb_v[k, pl.ds(g * L, L)] for g in range(CHUNK // L)]

        @plsc.parallel_loop(0, D, step=1, unroll=8)
        def cbody(c, k=k, cbs=cbs):
            for g in range(CHUNK // L):
                vals = plsc.load_gather(g_v.at[k], [rows[g], cbs[g] + c])
                h_v[k, c, pl.ds(g * L, L)] = (
                    h_v[k, c, pl.ds(g * L, L)] + vals)
        ocp.append(pltpu.async_copy(
            h_v.at[k], outT_hbm.at[:, pl.ds(base + k * CHUNK, CHUNK)], osem))

    for cp in ocp:
        cp.wait()


def kernel(head_embed, rel_ids, embed_table):
    aux = embed_table[NMAIN:].reshape(NTAILP, 128)
    s = _transpose_pairs(embed_table.T, aux)
    out_t = _gather_add(s, head_embed.T, rel_ids)
    return out_t.T
